# speedup vs baseline: 12.9157x; 12.9157x over previous
import jax
import jax.numpy as jnp
from jax import lax
from jax.experimental import pallas as pl
from jax.experimental.pallas import tpu as pltpu

MEM = 1024
D = 64
N = 16384
R = 1024
NB = N // R
SCALE = 1.25
NEG_INF = float("-inf")


def _pass_a(q_ref, keys_ref, sm_ref, uq_ref, qf_ref, m_ref, g_ref,
            cmax_ref, csum_ref, misc_ref):
    i = pl.program_id(0)
    q = q_ref[0]
    qt = q.reshape(D, R).T
    ss = jnp.sum(qt * qt, axis=1, keepdims=True)
    qf = qt / jnp.maximum(jnp.sqrt(ss), 1e-12)
    qf_ref[...] = qf
    keys = keys_ref[...]

    l = lax.dot_general(qf, keys, (((1,), (1,)), ((), ())),
                        preferred_element_type=jnp.float32) * SCALE
    m = jnp.max(l, axis=1)
    expl = jnp.exp(l - m[:, None])
    rs = jnp.sum(expl, axis=1)
    sm = expl * (1.0 / rs)[:, None]
    sm_ref[...] = sm

    cols = lax.broadcasted_iota(jnp.int32, (R, MEM), 1)
    gi = jnp.min(jnp.where(l == m[:, None], cols, MEM), axis=1)
    mask1 = cols == gi[:, None]
    l2 = jnp.where(mask1, NEG_INF, l)
    m2 = jnp.max(l2, axis=1)
    g2 = jnp.min(jnp.where(l2 == m2[:, None], cols, MEM), axis=1)
    mask2 = cols == g2[:, None]

    m_ref[...] = m[None, None, :]
    g_ref[...] = gi[None, None, :]

    pos = lax.dot_general(mask1.astype(jnp.float32), keys,
                          (((1,), (0,)), ((), ())),
                          preferred_element_type=jnp.float32)
    neg = lax.dot_general(mask2.astype(jnp.float32), keys,
                          (((1,), (0,)), ((), ())),
                          preferred_element_type=jnp.float32)
    cm = lax.dot_general(sm, keys, (((1,), (0,)), ((), ())),
                         preferred_element_type=jnp.float32)
    uq = jnp.concatenate([qf, cm], axis=1)
    uq_ref[...] = uq.T.reshape(1, 2 * D, 32, 32)

    comp_p = jnp.sum((qf - pos) ** 2)
    dp = jnp.sqrt(jnp.sum((qf - pos + 1e-6) ** 2, axis=1))
    dn = jnp.sqrt(jnp.sum((qf - neg + 1e-6) ** 2, axis=1))
    sep_p = jnp.sum(jnp.maximum(dp - dn + 1.0, 0.0))

    bmax = jnp.max(l, axis=0)[None, :]
    K = jnp.max(m)
    w = jnp.exp(m - K)
    bsum = lax.dot_general(w[None, :], expl, (((1,), (0,)), ((), ())),
                           preferred_element_type=jnp.float32)

    @pl.when(i == 0)
    def _():
        cmax_ref[...] = jnp.full((1, MEM), NEG_INF, jnp.float32)
        csum_ref[...] = jnp.zeros((1, MEM), jnp.float32)
        misc_ref[...] = jnp.zeros((1, 128), jnp.float32)

    old_m = cmax_ref[...]
    old_s = csum_ref[...]
    new_m = jnp.maximum(old_m, bmax)
    new_s = old_s * jnp.exp(old_m - new_m) + bsum * jnp.exp(K - new_m)
    cmax_ref[...] = new_m
    csum_ref[...] = new_s

    lanes = lax.broadcasted_iota(jnp.int32, (1, 128), 1)
    contrib = (jnp.where(lanes == 0, comp_p, 0.0)
               + jnp.where(lanes == 1, sep_p, 0.0))
    misc_ref[...] = misc_ref[...] + contrib

    @pl.when(i == NB - 1)
    def _():
        acc = misc_ref[...]
        scale_vec = jnp.where(lanes == 0, 1.0 / (N * D),
                              jnp.where(lanes == 1, 1.0 / N, 0.0))
        gmax = jnp.max(new_m)
        misc_ref[...] = acc * scale_vec + jnp.where(lanes == 2, gmax, 0.0)


def _pass_b(qf_ref, keys_ref, cmax_ref, csum_ref, sq_ref, um_ref):
    i = pl.program_id(0)
    qf = qf_ref[...]
    keys = keys_ref[...]
    l = lax.dot_general(qf, keys, (((1,), (1,)), ((), ())),
                        preferred_element_type=jnp.float32) * SCALE
    c_row = cmax_ref[...] + jnp.log(csum_ref[...])
    sq = jnp.exp(l - c_row)
    sq_ref[...] = sq

    m = jnp.max(l, axis=1)
    cols = lax.broadcasted_iota(jnp.int32, (R, MEM), 1)
    gi = jnp.min(jnp.where(l == m[:, None], cols, MEM), axis=1)
    mask1 = cols == gi[:, None]
    W = jnp.where(mask1, sq, 0.0)
    contrib = lax.dot_general(W, qf, (((0,), (0,)), ((), ())),
                              preferred_element_type=jnp.float32)

    @pl.when(i == 0)
    def _():
        um_ref[...] = jnp.zeros((MEM, D), jnp.float32)

    um_ref[...] = um_ref[...] + contrib

    @pl.when(i == NB - 1)
    def _():
        qu = um_ref[...]
        um = 0.5 * keys + 0.5 * qu
        nrm = jnp.sqrt(jnp.sum(um * um, axis=1, keepdims=True))
        um_ref[...] = um / jnp.maximum(nrm, 1e-12)


def kernel(query, keys):
    b, dims, h, w = query.shape

    sm, uq, qf, m3, g3, cmax, csum, misc = pl.pallas_call(
        _pass_a,
        grid=(NB,),
        in_specs=[
            pl.BlockSpec((1, D, 32, 32), lambda i: (i, 0, 0, 0)),
            pl.BlockSpec((MEM, D), lambda i: (0, 0)),
        ],
        out_specs=[
            pl.BlockSpec((R, MEM), lambda i: (i, 0)),
            pl.BlockSpec((1, 2 * D, 32, 32), lambda i: (i, 0, 0, 0)),
            pl.BlockSpec((R, D), lambda i: (i, 0)),
            pl.BlockSpec((1, 1, R), lambda i: (i, 0, 0)),
            pl.BlockSpec((1, 1, R), lambda i: (i, 0, 0)),
            pl.BlockSpec((1, MEM), lambda i: (0, 0)),
            pl.BlockSpec((1, MEM), lambda i: (0, 0)),
            pl.BlockSpec((1, 128), lambda i: (0, 0)),
        ],
        out_shape=[
            jax.ShapeDtypeStruct((N, MEM), jnp.float32),
            jax.ShapeDtypeStruct((b, 2 * D, h, w), jnp.float32),
            jax.ShapeDtypeStruct((N, D), jnp.float32),
            jax.ShapeDtypeStruct((NB, 1, R), jnp.float32),
            jax.ShapeDtypeStruct((NB, 1, R), jnp.int32),
            jax.ShapeDtypeStruct((1, MEM), jnp.float32),
            jax.ShapeDtypeStruct((1, MEM), jnp.float32),
            jax.ShapeDtypeStruct((1, 128), jnp.float32),
        ],
    )(query, keys)

    sq, um = pl.pallas_call(
        _pass_b,
        grid=(NB,),
        in_specs=[
            pl.BlockSpec((R, D), lambda i: (i, 0)),
            pl.BlockSpec((MEM, D), lambda i: (0, 0)),
            pl.BlockSpec((1, MEM), lambda i: (0, 0)),
            pl.BlockSpec((1, MEM), lambda i: (0, 0)),
        ],
        out_specs=[
            pl.BlockSpec((R, MEM), lambda i: (i, 0)),
            pl.BlockSpec((MEM, D), lambda i: (0, 0)),
        ],
        out_shape=[
            jax.ShapeDtypeStruct((N, MEM), jnp.float32),
            jax.ShapeDtypeStruct((MEM, D), jnp.float32),
        ],
    )(qf, keys, cmax, csum)

    comp = misc[0, 0]
    sep = misc[0, 1]
    return (uq, um, sq, sm, sep, comp)
